# Initial kernel scaffold; baseline (speedup 1.0000x reference)
import jax
import jax.numpy as jnp
from jax.experimental import pallas as pl


def kernel(x, weight_q, weight_r):
    raise NotImplementedError("write your pallas kernel here")


# speedup vs baseline: 6.9835x; 6.9835x over previous
import functools

import jax
import jax.numpy as jnp
from jax import lax
from jax.experimental import pallas as pl
from jax.experimental.pallas import tpu as pltpu
from jax.experimental.pallas import tpu_sc as plsc

_NUM_BUCKETS = 1001
_D = 16
_NC = 2
_NS = 16
_NW = _NC * _NS
_RECIP = 1.0 / _NUM_BUCKETS


def _qr_body(x_hbm, wq_hbm, wr_hbm, out_hbm, wq_v, wr_v, xv, obuf,
             *, per_w, ch, nch):
    wid = lax.axis_index("s") * _NC + lax.axis_index("c")
    base = pl.multiple_of(wid * per_w, 8)
    pltpu.sync_copy(wq_hbm, wq_v)
    pltpu.sync_copy(wr_hbm, wr_v)
    pltpu.sync_copy(x_hbm.at[pl.ds(base, per_w)], xv)
    lanes = lax.broadcasted_iota(jnp.int32, (16,), 0)

    def chunk_body(c, carry):
        row0 = c * ch

        def grp(g, carry2):
            off = pl.multiple_of(row0 + g * 16, 8)
            v = xv[pl.ds(off, 16)]
            q = (v.astype(jnp.float32) * _RECIP).astype(jnp.int32)
            r = v - q * _NUM_BUCKETS
            neg = r < 0
            q = jnp.where(neg, q - 1, q)
            r = jnp.where(neg, r + _NUM_BUCKETS, r)
            big = r >= _NUM_BUCKETS
            q = jnp.where(big, q + 1, q)
            r = jnp.where(big, r - _NUM_BUCKETS, r)
            q16 = q * _D
            r16 = r * _D
            rows16 = (g * 16 + lanes) * _D
            for d in range(_D):
                qe = plsc.load_gather(wq_v, [q16 + d])
                re = plsc.load_gather(wr_v, [r16 + d])
                plsc.store_scatter(obuf, [rows16 + d], qe * re)
            return carry2

        lax.fori_loop(0, ch // 16, grp, 0)
        dst = pl.multiple_of((base + row0) * _D, 8)
        pltpu.sync_copy(obuf, out_hbm.at[pl.ds(dst, ch * _D)])
        return carry

    lax.fori_loop(0, nch, chunk_body, 0)


def kernel(x, weight_q, weight_r):
    B, F = x.shape
    n = B * F
    per_w = n // _NW
    ch = 1024
    nch = per_w // ch
    assert per_w * _NW == n and nch * ch == per_w
    xf = x.reshape(n)
    mesh = plsc.VectorSubcoreMesh(core_axis_name="c", subcore_axis_name="s")
    body = functools.partial(_qr_body, per_w=per_w, ch=ch, nch=nch)
    out = pl.kernel(
        body,
        out_type=jax.ShapeDtypeStruct((n * _D,), jnp.float32),
        mesh=mesh,
        compiler_params=pltpu.CompilerParams(needs_layout_passes=False),
        scratch_types=[
            pltpu.VMEM((_NUM_BUCKETS * _D,), jnp.float32),
            pltpu.VMEM((_NUM_BUCKETS * _D,), jnp.float32),
            pltpu.VMEM((per_w,), jnp.int32),
            pltpu.VMEM((ch * _D,), jnp.float32),
        ],
    )(xf.reshape(n), weight_q.reshape(_NUM_BUCKETS * _D),
      weight_r.reshape(_NUM_BUCKETS * _D))
    return out.reshape(B, F, _D)


# speedup vs baseline: 9.0530x; 1.2963x over previous
import functools

import jax
import jax.numpy as jnp
from jax import lax
from jax.experimental import pallas as pl
from jax.experimental.pallas import tpu as pltpu
from jax.experimental.pallas import tpu_sc as plsc

_NUM_BUCKETS = 1001
_D = 16
_NC = 2
_NS = 16
_NW = _NC * _NS
_RECIP = 1.0 / _NUM_BUCKETS


def _qr_body(x_hbm, wq_hbm, wr_hbm, out_hbm, wq_v, wr_v, xv, obuf_a, obuf_b,
             sem_a, sem_b, *, per_w, ch, nch, unroll):
    wid = lax.axis_index("s") * _NC + lax.axis_index("c")
    base = pl.multiple_of(wid * per_w, 8)
    pltpu.sync_copy(wq_hbm, wq_v)
    pltpu.sync_copy(wr_hbm, wr_v)
    pltpu.sync_copy(x_hbm.at[pl.ds(base, per_w)], xv)
    lanes = lax.broadcasted_iota(jnp.int32, (16,), 0)

    bufs = (obuf_a, obuf_b)
    sems = (sem_a, sem_b)
    pending = [None, None]
    for c in range(nch):
        s = c % 2
        obuf = bufs[s]
        if pending[s] is not None:
            pending[s].wait()
        row0 = c * ch

        @plsc.parallel_loop(0, ch // 16, unroll=unroll)
        def _(g):
            off = pl.multiple_of(row0 + g * 16, 8)
            v = xv[pl.ds(off, 16)]
            q = (v.astype(jnp.float32) * _RECIP).astype(jnp.int32)
            r = v - q * _NUM_BUCKETS
            neg = r < 0
            q = jnp.where(neg, q - 1, q)
            r = jnp.where(neg, r + _NUM_BUCKETS, r)
            big = r >= _NUM_BUCKETS
            q = jnp.where(big, q + 1, q)
            r = jnp.where(big, r - _NUM_BUCKETS, r)
            q16 = q * _D
            r16 = r * _D
            rows16 = (g * 16 + lanes) * _D
            for d in range(_D):
                qe = plsc.load_gather(wq_v, [q16 + d])
                re = plsc.load_gather(wr_v, [r16 + d])
                plsc.store_scatter(obuf, [rows16 + d], qe * re)

        dst = pl.multiple_of((base + row0) * _D, 8)
        pending[s] = pltpu.async_copy(
            obuf, out_hbm.at[pl.ds(dst, ch * _D)], sems[s])
    for p in pending:
        if p is not None:
            p.wait()


def kernel(x, weight_q, weight_r):
    B, F = x.shape
    n = B * F
    per_w = n // _NW
    ch = 1024
    nch = per_w // ch
    assert per_w * _NW == n and nch * ch == per_w
    mesh = plsc.VectorSubcoreMesh(core_axis_name="c", subcore_axis_name="s")
    body = functools.partial(_qr_body, per_w=per_w, ch=ch, nch=nch, unroll=2)
    out = pl.kernel(
        body,
        out_type=jax.ShapeDtypeStruct((n * _D,), jnp.float32),
        mesh=mesh,
        compiler_params=pltpu.CompilerParams(needs_layout_passes=False),
        scratch_types=[
            pltpu.VMEM((_NUM_BUCKETS * _D,), jnp.float32),
            pltpu.VMEM((_NUM_BUCKETS * _D,), jnp.float32),
            pltpu.VMEM((per_w,), jnp.int32),
            pltpu.VMEM((ch * _D,), jnp.float32),
            pltpu.VMEM((ch * _D,), jnp.float32),
            pltpu.SemaphoreType.DMA,
            pltpu.SemaphoreType.DMA,
        ],
    )(x.reshape(n), weight_q.reshape(_NUM_BUCKETS * _D),
      weight_r.reshape(_NUM_BUCKETS * _D))
    return out.reshape(B, F, _D)


# speedup vs baseline: 21.6892x; 2.3958x over previous
import functools

import jax
import jax.numpy as jnp
from jax import lax
from jax.experimental import pallas as pl
from jax.experimental.pallas import tpu as pltpu
from jax.experimental.pallas import tpu_sc as plsc

_NUM_BUCKETS = 1001
_D = 16
_NC = 2
_NS = 16
_NW = _NC * _NS
_RECIP = 1.0 / _NUM_BUCKETS


def _divmod_const(v, div, recip):
    q = (v.astype(jnp.float32) * recip).astype(jnp.int32)
    r = v - q * div
    neg = r < 0
    q = jnp.where(neg, q - 1, q)
    r = jnp.where(neg, r + div, r)
    big = r >= div
    q = jnp.where(big, q + 1, q)
    r = jnp.where(big, r - div, r)
    return q, r


def _qr_body(x_hbm, wq_hbm, wr_hbm, out_hbm, wq_v, wr_v, xv, lbuf_a, lbuf_b,
             sem_a, sem_b, *, nfields, per_wb, bt_chunk, fh, unroll):
    per_w = per_wb * nfields
    wid = lax.axis_index("s") * _NC + lax.axis_index("c")
    b0 = pl.multiple_of(wid * per_wb, 128)
    base = pl.multiple_of(wid * per_w, 8)
    pltpu.sync_copy(wq_hbm, wq_v)
    pltpu.sync_copy(wr_hbm, wr_v)
    pltpu.sync_copy(x_hbm.at[pl.ds(base, per_w)], xv)
    lanes = lax.broadcasted_iota(jnp.int32, (16,), 0)

    bufs = (lbuf_a, lbuf_b)
    sems = (sem_a, sem_b)
    pending = [[], []]
    n_bt = per_wb // bt_chunk
    chunk_id = 0
    for bt in range(n_bt):
        bc = bt * bt_chunk
        for f0 in range(0, nfields, fh):
            s = chunk_id % 2
            chunk_id += 1
            lbuf = bufs[s]
            for p in pending[s]:
                p.wait()
            pending[s] = []

            def f_body(f_rel, _, *, f0=f0, bc=bc, lbuf=lbuf):
                f = f0 + f_rel

                @plsc.parallel_loop(0, bt_chunk // 16, unroll=unroll)
                def _(g):
                    bl0 = g * 16
                    xidx = (bc + bl0 + lanes) * nfields + f
                    v = plsc.load_gather(xv, [xidx])
                    q, r = _divmod_const(v, _NUM_BUCKETS, _RECIP)
                    q16 = q * _D
                    r16 = r * _D
                    for d in range(_D):
                        qe = plsc.load_gather(wq_v, [q16 + d])
                        re = plsc.load_gather(wr_v, [r16 + d])
                        lbuf[f_rel, d, pl.ds(bl0, 16)] = qe * re
                return 0

            lax.fori_loop(0, fh, f_body, 0)
            for f_rel in range(fh):
                pending[s].append(pltpu.async_copy(
                    lbuf.at[f_rel],
                    out_hbm.at[f0 + f_rel, :, pl.ds(b0 + bc, bt_chunk)],
                    sems[s]))
    for plist in pending:
        for p in plist:
            p.wait()


def kernel(x, weight_q, weight_r):
    B, F = x.shape
    n = B * F
    per_wb = B // _NW
    bt_chunk = 128
    fh = 13
    assert per_wb * _NW == B and per_wb % bt_chunk == 0 and F % fh == 0
    mesh = plsc.VectorSubcoreMesh(core_axis_name="c", subcore_axis_name="s")
    body = functools.partial(_qr_body, nfields=F, per_wb=per_wb,
                             bt_chunk=bt_chunk, fh=fh, unroll=2)
    out = pl.kernel(
        body,
        out_type=jax.ShapeDtypeStruct((F, _D, B), jnp.float32),
        mesh=mesh,
        compiler_params=pltpu.CompilerParams(needs_layout_passes=False),
        scratch_types=[
            pltpu.VMEM((_NUM_BUCKETS * _D,), jnp.float32),
            pltpu.VMEM((_NUM_BUCKETS * _D,), jnp.float32),
            pltpu.VMEM((per_wb * F,), jnp.int32),
            pltpu.VMEM((fh, _D, bt_chunk), jnp.float32),
            pltpu.VMEM((fh, _D, bt_chunk), jnp.float32),
            pltpu.SemaphoreType.DMA,
            pltpu.SemaphoreType.DMA,
        ],
    )(x.reshape(n), weight_q.reshape(_NUM_BUCKETS * _D),
      weight_r.reshape(_NUM_BUCKETS * _D))
    return out.transpose(2, 0, 1)


# speedup vs baseline: 21.6975x; 1.0004x over previous
import functools

import jax
import jax.numpy as jnp
from jax import lax
from jax.experimental import pallas as pl
from jax.experimental.pallas import tpu as pltpu
from jax.experimental.pallas import tpu_sc as plsc

_NUM_BUCKETS = 1001
_D = 16
_NC = 2
_NS = 16
_NW = _NC * _NS
_RECIP = 1.0 / _NUM_BUCKETS


def _divmod_buckets(v):
    q = (v.astype(jnp.float32) * _RECIP).astype(jnp.int32)
    r = v - q * _NUM_BUCKETS
    big = r >= _NUM_BUCKETS
    q = jnp.where(big, q + 1, q)
    r = jnp.where(big, r - _NUM_BUCKETS, r)
    return q, r


def _qr_body(xt_hbm, wq_hbm, wr_hbm, out_hbm, wq_v, wr_v, xv, lbuf_a, lbuf_b,
             sem_a, sem_b, *, nfields, per_wb, bt_chunk, fh, unroll):
    wid = lax.axis_index("s") * _NC + lax.axis_index("c")
    b0 = pl.multiple_of(wid * per_wb, 128)
    pltpu.sync_copy(wq_hbm, wq_v)
    pltpu.sync_copy(wr_hbm, wr_v)
    n_bt = per_wb // bt_chunk
    for bt in range(n_bt):
        for f8 in range(0, nfields - 7, 8):
            pltpu.sync_copy(
                xt_hbm.at[pl.ds(f8, 8), pl.ds(b0 + bt * bt_chunk, bt_chunk)],
                xv.at[pl.ds(f8, 8), bt, :])
        rem = nfields % 8
        if rem:
            f8 = nfields - rem
            pltpu.sync_copy(
                xt_hbm.at[pl.ds(f8, rem), pl.ds(b0 + bt * bt_chunk, bt_chunk)],
                xv.at[pl.ds(f8, rem), bt, :])
    lanes = lax.broadcasted_iota(jnp.int32, (16,), 0)
    del lanes

    bufs = (lbuf_a, lbuf_b)
    sems = (sem_a, sem_b)
    pending = [[], []]
    g_per_f = bt_chunk // 16
    chunk_id = 0
    for bt in range(n_bt):
        for f0 in range(0, nfields, fh):
            s = chunk_id % 2
            chunk_id += 1
            lbuf = bufs[s]
            for p in pending[s]:
                p.wait()
            pending[s] = []

            @plsc.parallel_loop(0, fh * g_per_f, unroll=unroll)
            def _(t, *, f0=f0, bt=bt, lbuf=lbuf):
                f_rel = t >> 3
                bl0 = (t & (g_per_f - 1)) * 16
                v = xv[f0 + f_rel, bt, pl.ds(bl0, 16)]
                q, r = _divmod_buckets(v)
                q16 = q * _D
                r16 = r * _D
                for d in range(_D):
                    qe = plsc.load_gather(wq_v, [q16 + d])
                    re = plsc.load_gather(wr_v, [r16 + d])
                    lbuf[f_rel, d, pl.ds(bl0, 16)] = qe * re

            for f_rel in range(fh):
                pending[s].append(pltpu.async_copy(
                    lbuf.at[f_rel],
                    out_hbm.at[f0 + f_rel, :,
                               pl.ds(b0 + bt * bt_chunk, bt_chunk)],
                    sems[s]))
    for plist in pending:
        for p in plist:
            p.wait()


def kernel(x, weight_q, weight_r):
    B, F = x.shape
    per_wb = B // _NW
    bt_chunk = 128
    fh = 13
    assert per_wb * _NW == B and per_wb % bt_chunk == 0 and F % fh == 0
    assert bt_chunk // 16 == 8
    mesh = plsc.VectorSubcoreMesh(core_axis_name="c", subcore_axis_name="s")
    body = functools.partial(_qr_body, nfields=F, per_wb=per_wb,
                             bt_chunk=bt_chunk, fh=fh, unroll=4)
    out = pl.kernel(
        body,
        out_type=jax.ShapeDtypeStruct((F, _D, B), jnp.float32),
        mesh=mesh,
        compiler_params=pltpu.CompilerParams(needs_layout_passes=False),
        scratch_types=[
            pltpu.VMEM((_NUM_BUCKETS * _D,), jnp.float32),
            pltpu.VMEM((_NUM_BUCKETS * _D,), jnp.float32),
            pltpu.VMEM((F, per_wb // bt_chunk, bt_chunk), jnp.int32),
            pltpu.VMEM((fh, _D, bt_chunk), jnp.float32),
            pltpu.VMEM((fh, _D, bt_chunk), jnp.float32),
            pltpu.SemaphoreType.DMA,
            pltpu.SemaphoreType.DMA,
        ],
    )(x.T, weight_q.reshape(_NUM_BUCKETS * _D),
      weight_r.reshape(_NUM_BUCKETS * _D))
    return out.transpose(2, 0, 1)


# speedup vs baseline: 40.5965x; 1.8710x over previous
import functools

import jax
import jax.numpy as jnp
from jax import lax
from jax.experimental import pallas as pl
from jax.experimental.pallas import tpu as pltpu
from jax.experimental.pallas import tpu_sc as plsc

_NUM_BUCKETS = 1001
_D = 16
_NC = 2
_NS = 16
_NW = _NC * _NS
_RECIP = 1.0 / _NUM_BUCKETS


def _divmod_buckets(v):
    q = (v.astype(jnp.float32) * _RECIP).astype(jnp.int32)
    r = v - q * _NUM_BUCKETS
    big = r >= _NUM_BUCKETS
    q = jnp.where(big, q + 1, q)
    r = jnp.where(big, r - _NUM_BUCKETS, r)
    return q, r


def _qr_body(xt_hbm, wq_hbm, wr_hbm, out_hbm, wq_v, wr_v, xv, lbuf_a, lbuf_b,
             sem_a, sem_b, *, nfields, per_wb, bt_chunk, fh, unroll):
    wid = lax.axis_index("s") * _NC + lax.axis_index("c")
    b0 = pl.multiple_of(wid * per_wb, 128)
    pltpu.sync_copy(wq_hbm, wq_v)
    pltpu.sync_copy(wr_hbm, wr_v)
    n_bt = per_wb // bt_chunk
    for bt in range(n_bt):
        for f8 in range(0, nfields - 7, 8):
            pltpu.sync_copy(
                xt_hbm.at[pl.ds(f8, 8), pl.ds(b0 + bt * bt_chunk, bt_chunk)],
                xv.at[pl.ds(f8, 8), bt, :])
        rem = nfields % 8
        if rem:
            f8 = nfields - rem
            pltpu.sync_copy(
                xt_hbm.at[pl.ds(f8, rem), pl.ds(b0 + bt * bt_chunk, bt_chunk)],
                xv.at[pl.ds(f8, rem), bt, :])
    bufs = (lbuf_a, lbuf_b)
    sems = (sem_a, sem_b)
    pending = [[], []]
    g_per_f = bt_chunk // 16
    chunk_id = 0
    for bt in range(n_bt):
        for f0 in range(0, nfields, fh):
            s = chunk_id % 2
            chunk_id += 1
            lbuf = bufs[s]
            for p in pending[s]:
                p.wait()
            pending[s] = []

            @plsc.parallel_loop(0, fh * g_per_f, unroll=unroll)
            def _(t, *, f0=f0, bt=bt, lbuf=lbuf):
                f_rel = t >> 3
                bl0 = (t & (g_per_f - 1)) * 16
                v = xv[f0 + f_rel, bt, pl.ds(bl0, 16)]
                q, r = _divmod_buckets(v)
                for d in range(_D):
                    qe = plsc.load_gather(wq_v, [q + d * _NUM_BUCKETS])
                    re = plsc.load_gather(wr_v, [r + d * _NUM_BUCKETS])
                    lbuf[f_rel, d, pl.ds(bl0, 16)] = qe * re

            for f_rel in range(fh):
                pending[s].append(pltpu.async_copy(
                    lbuf.at[f_rel],
                    out_hbm.at[f0 + f_rel, :,
                               pl.ds(b0 + bt * bt_chunk, bt_chunk)],
                    sems[s]))
    for plist in pending:
        for p in plist:
            p.wait()


def kernel(x, weight_q, weight_r):
    B, F = x.shape
    per_wb = B // _NW
    bt_chunk = 128
    fh = 13
    assert per_wb * _NW == B and per_wb % bt_chunk == 0 and F % fh == 0
    assert bt_chunk // 16 == 8
    mesh = plsc.VectorSubcoreMesh(core_axis_name="c", subcore_axis_name="s")
    body = functools.partial(_qr_body, nfields=F, per_wb=per_wb,
                             bt_chunk=bt_chunk, fh=fh, unroll=4)
    out = pl.kernel(
        body,
        out_type=jax.ShapeDtypeStruct((F, _D, B), jnp.float32),
        mesh=mesh,
        compiler_params=pltpu.CompilerParams(needs_layout_passes=False),
        scratch_types=[
            pltpu.VMEM((_NUM_BUCKETS * _D,), jnp.float32),
            pltpu.VMEM((_NUM_BUCKETS * _D,), jnp.float32),
            pltpu.VMEM((F, per_wb // bt_chunk, bt_chunk), jnp.int32),
            pltpu.VMEM((fh, _D, bt_chunk), jnp.float32),
            pltpu.VMEM((fh, _D, bt_chunk), jnp.float32),
            pltpu.SemaphoreType.DMA,
            pltpu.SemaphoreType.DMA,
        ],
    )(x.T, weight_q.T.reshape(_NUM_BUCKETS * _D),
      weight_r.T.reshape(_NUM_BUCKETS * _D))
    return out.transpose(2, 0, 1)
